# baseline (speedup 1.0000x reference)
import jax
import jax.numpy as jnp
from jax.experimental import pallas as pl
from jax.experimental.pallas import tpu as pltpu

_N = 10000
_H = 128
_BM = 200


def _layer1_body(adj_ref, x_ref, W1_ref, b_ref, out_ref, s_ref):
    @pl.when(pl.program_id(0) == 0)
    def _():
        s_ref[...] = jnp.dot(x_ref[...], W1_ref[...],
                             preferred_element_type=jnp.float32)

    acc = jnp.dot(adj_ref[...], s_ref[...],
                  preferred_element_type=jnp.float32)
    out_ref[...] = jnp.maximum(acc + b_ref[...], 0.0)


def _adj_layer1(adj, x, W1, b):
    full = lambda i: (0, 0)
    return pl.pallas_call(
        _layer1_body,
        grid=(_N // _BM,),
        in_specs=[
            pl.BlockSpec((_BM, _N), lambda i: (i, 0)),
            pl.BlockSpec((_N, _H), full),
            pl.BlockSpec((_H, _H), full),
            pl.BlockSpec((1, _H), full),
        ],
        out_specs=pl.BlockSpec((_BM, _H), lambda i: (i, 0)),
        out_shape=jax.ShapeDtypeStruct((_N, _H), jnp.float32),
        scratch_shapes=[pltpu.VMEM((_N, _H), jnp.float32)],
        compiler_params=pltpu.CompilerParams(
            dimension_semantics=("arbitrary",)),
    )(adj, x, W1, b)


def _layer2_body(adj_ref, h1_ref, W2_ref, b2_ref, Wt1_ref, bt1_ref, Wt2_ref,
                 bt2_ref, Wp_ref, bp_ref, rep_ref, tau_ref, e_ref, s_ref):
    @pl.when(pl.program_id(0) == 0)
    def _():
        s_ref[...] = jnp.dot(h1_ref[...], W2_ref[...],
                             preferred_element_type=jnp.float32)

    acc = jnp.dot(adj_ref[...], s_ref[...],
                  preferred_element_type=jnp.float32)
    h2 = jnp.maximum(acc + b2_ref[...], 0.0)
    rep_ref[...] = h2
    t = jnp.maximum(
        jnp.dot(h2, Wt1_ref[...], preferred_element_type=jnp.float32)
        + bt1_ref[...], 0.0)
    tau_ref[...] = (jnp.dot(t, Wt2_ref[...], preferred_element_type=jnp.float32)
                    + bt2_ref[...])
    e_ref[...] = jax.nn.sigmoid(
        jnp.dot(h2, Wp_ref[...], preferred_element_type=jnp.float32)
        + bp_ref[...])


def _adj_layer2(adj, h1, W2, b2, Wt1, bt1, Wt2, bt2, Wp, bp):
    full = lambda i: (0, 0)
    return pl.pallas_call(
        _layer2_body,
        grid=(_N // _BM,),
        in_specs=[
            pl.BlockSpec((_BM, _N), lambda i: (i, 0)),
            pl.BlockSpec((_N, _H), full),
            pl.BlockSpec((_H, _H), full),
            pl.BlockSpec((1, _H), full),
            pl.BlockSpec((_H, _H), full),
            pl.BlockSpec((1, _H), full),
            pl.BlockSpec((_H, 1), full),
            pl.BlockSpec((1, 1), full),
            pl.BlockSpec((_H, 1), full),
            pl.BlockSpec((1, 1), full),
        ],
        out_specs=[
            pl.BlockSpec((_BM, _H), lambda i: (i, 0)),
            pl.BlockSpec((_BM, 1), lambda i: (i, 0)),
            pl.BlockSpec((_BM, 1), lambda i: (i, 0)),
        ],
        out_shape=[
            jax.ShapeDtypeStruct((_N, _H), jnp.float32),
            jax.ShapeDtypeStruct((_N, 1), jnp.float32),
            jax.ShapeDtypeStruct((_N, 1), jnp.float32),
        ],
        scratch_shapes=[pltpu.VMEM((_N, _H), jnp.float32)],
        compiler_params=pltpu.CompilerParams(
            dimension_semantics=("arbitrary",)),
    )(adj, h1, W2, b2, Wt1, bt1, Wt2, bt2, Wp, bp)


def kernel(x, adj, W1, b1, W2, b2, Wt1, bt1, Wt2, bt2, Wp, bp):
    h1 = _adj_layer1(adj, x, W1, b1.reshape(1, _H))
    rep, tau, e = _adj_layer2(adj, h1, W2, b2.reshape(1, _H), Wt1,
                              bt1.reshape(1, _H), Wt2, bt2.reshape(1, 1),
                              Wp, bp.reshape(1, 1))
    tau = tau[:, 0]
    e = e[:, 0]
    z = jnp.zeros_like(tau)
    return (e, z, tau, tau, tau, z, z, rep)


# speedup vs baseline: 1.0378x; 1.0378x over previous
import jax
import jax.numpy as jnp
from jax.experimental import pallas as pl
from jax.experimental.pallas import tpu as pltpu

_N = 10000
_H = 128
_BM = 400


def _body(adj_ref, x_ref, W1_ref, b1_ref, W2_ref, b2_ref, Wt1_ref, bt1_ref,
          Wt2_ref, bt2_ref, Wp_ref, bp_ref, rep_ref, tau_ref, e_ref,
          s_ref, h1_ref):
    p = pl.program_id(0)
    i = pl.program_id(1)

    @pl.when((p == 0) & (i == 0))
    def _():
        s_ref[...] = jnp.dot(x_ref[...], W1_ref[...],
                             preferred_element_type=jnp.float32)

    @pl.when((p == 1) & (i == 0))
    def _():
        s_ref[...] = jnp.dot(h1_ref[...], W2_ref[...],
                             preferred_element_type=jnp.float32)

    acc = jnp.dot(adj_ref[...], s_ref[...],
                  preferred_element_type=jnp.float32)

    @pl.when(p == 0)
    def _():
        h1_ref[pl.ds(i * _BM, _BM), :] = jnp.maximum(acc + b1_ref[...], 0.0)

    @pl.when(p == 1)
    def _():
        h2 = jnp.maximum(acc + b2_ref[...], 0.0)
        rep_ref[...] = h2
        t = jnp.maximum(
            jnp.dot(h2, Wt1_ref[...], preferred_element_type=jnp.float32)
            + bt1_ref[...], 0.0)
        tau_ref[...] = (jnp.dot(t, Wt2_ref[...],
                                preferred_element_type=jnp.float32)
                        + bt2_ref[...])
        e_ref[...] = jax.nn.sigmoid(
            jnp.dot(h2, Wp_ref[...], preferred_element_type=jnp.float32)
            + bp_ref[...])


def kernel(x, adj, W1, b1, W2, b2, Wt1, bt1, Wt2, bt2, Wp, bp):
    full = lambda p, i: (0, 0)
    out_idx = lambda p, i: (jnp.where(p == 0, 0, i), 0)
    rep, tau, e = pl.pallas_call(
        _body,
        grid=(2, _N // _BM),
        in_specs=[
            pl.BlockSpec((_BM, _N), lambda p, i: (i, 0)),
            pl.BlockSpec((_N, _H), full),
            pl.BlockSpec((_H, _H), full),
            pl.BlockSpec((1, _H), full),
            pl.BlockSpec((_H, _H), full),
            pl.BlockSpec((1, _H), full),
            pl.BlockSpec((_H, _H), full),
            pl.BlockSpec((1, _H), full),
            pl.BlockSpec((_H, 1), full),
            pl.BlockSpec((1, 1), full),
            pl.BlockSpec((_H, 1), full),
            pl.BlockSpec((1, 1), full),
        ],
        out_specs=[
            pl.BlockSpec((_BM, _H), out_idx),
            pl.BlockSpec((_BM, 1), out_idx),
            pl.BlockSpec((_BM, 1), out_idx),
        ],
        out_shape=[
            jax.ShapeDtypeStruct((_N, _H), jnp.float32),
            jax.ShapeDtypeStruct((_N, 1), jnp.float32),
            jax.ShapeDtypeStruct((_N, 1), jnp.float32),
        ],
        scratch_shapes=[
            pltpu.VMEM((_N, _H), jnp.float32),
            pltpu.VMEM((_N, _H), jnp.float32),
        ],
        compiler_params=pltpu.CompilerParams(
            dimension_semantics=("arbitrary", "arbitrary")),
    )(adj, x, W1, b1.reshape(1, _H), W2, b2.reshape(1, _H), Wt1,
      bt1.reshape(1, _H), Wt2, bt2.reshape(1, 1), Wp, bp.reshape(1, 1))
    tau = tau[:, 0]
    e = e[:, 0]
    z = jnp.zeros_like(tau)
    return (e, z, tau, tau, tau, z, z, rep)
